# baseline (speedup 1.0000x reference)
import functools

import jax
import jax.numpy as jnp
from jax import lax
from jax.experimental import pallas as pl
from jax.experimental.pallas import tpu as pltpu

_LN_EPS = 1e-5


def _fused_kernel(x_ref, w_ref, b_ref, o_ref, s_ref, *, inv_e, tn):
    params = b_ref[...]
    for t in range(tn):
        xc = x_ref[t].astype(jnp.bfloat16)
        xc = xc.reshape(42, 16, 224)
        xb = jnp.stack([jnp.transpose(xc[i], (1, 0)) for i in range(42)])
        xb = xb.reshape(3, 14, 14, 16, 16)
        for pw in range(16):
            s_ref[:, :, 0:14, pw * 16:(pw + 1) * 16] = xb[:, :, :, pw, :]
        pat = s_ref[...].reshape(3, 224, 256)
        acc = jnp.dot(pat[0], w_ref[0], preferred_element_type=jnp.float32)
        acc = acc + jnp.dot(pat[1], w_ref[1],
                            preferred_element_type=jnp.float32)
        acc = acc + jnp.dot(pat[2], w_ref[2],
                            preferred_element_type=jnp.float32)
        acc = acc + params[0:1]
        mean = jnp.sum(acc, axis=-1, keepdims=True) * inv_e
        sumsq = jnp.sum(acc * acc, axis=-1, keepdims=True) * inv_e
        var = jnp.maximum(sumsq - mean * mean, 0.0)
        normed = (acc - mean) * lax.rsqrt(var + _LN_EPS)
        out = normed * params[1:2] + params[2:3]
        out_t = jnp.transpose(out, (1, 0)).astype(o_ref.dtype)
        for hp in range(14):
            o_ref[t, :, hp * 14:(hp + 1) * 14] = out_t[:, hp * 16:hp * 16 + 14]


def kernel(x, conv_w, conv_b, ln_g, ln_b):
    N, C, H, W = x.shape
    E = conv_w.shape[0]
    P = 16
    Hp, Wp = H // P, W // P
    HW = Hp * Wp
    K = C * P * P

    w_r = jnp.transpose(conv_w, (1, 3, 2, 0))
    w_r = w_r.reshape(C, P * P, E).astype(jnp.bfloat16)
    params = jnp.stack([conv_b, ln_g, ln_b]).astype(jnp.float32)
    params = jnp.pad(params, ((0, 8 - 3), (0, 0)))

    tn = 4
    grid = (N // tn,)
    cost = pl.CostEstimate(
        flops=2 * N * HW * K * E,
        transcendentals=N * HW,
        bytes_accessed=(N * C * H * W * 4 + C * P * P * E * 2 + 8 * E * 4
                        + N * E * HW * 4))

    out = pl.pallas_call(
        functools.partial(_fused_kernel, inv_e=1.0 / E, tn=tn),
        out_shape=jax.ShapeDtypeStruct((N, E, HW), x.dtype),
        grid=grid,
        in_specs=[
            pl.BlockSpec((tn, C, H, W), lambda i: (i, 0, 0, 0)),
            pl.BlockSpec((C, P * P, E), lambda i: (0, 0, 0)),
            pl.BlockSpec((8, E), lambda i: (0, 0)),
        ],
        out_specs=pl.BlockSpec((tn, E, HW), lambda i: (i, 0, 0)),
        scratch_shapes=[pltpu.VMEM((C, Hp, 16, P * P), jnp.bfloat16)],
        compiler_params=pltpu.CompilerParams(
            dimension_semantics=("parallel",),
            vmem_limit_bytes=96 * 1024 * 1024),
        cost_estimate=cost,
    )(x, w_r, params)

    return out.reshape(N, E, Hp, Wp)
